# Initial kernel scaffold; baseline (speedup 1.0000x reference)
import jax
import jax.numpy as jnp
from jax.experimental import pallas as pl


def kernel(target_nodes, features, edge3_others, edge2_others):
    raise NotImplementedError("write your pallas kernel here")


# speedup vs baseline: 10.2938x; 10.2938x over previous
import functools
import math

import jax
import jax.numpy as jnp
from jax import lax
from jax.experimental import pallas as pl
from jax.experimental.pallas import tpu as pltpu
from jax.experimental.pallas import tpu_sc as plsc

B = 8192
D3 = 12
D2 = 4
M = 3
D_FEAT = 256
DEG = D3 + D2


def _adj(c):
    alpha = 0
    for i in range(c):
        alpha += (-1) ** i * math.comb(c, i) * (c - i) ** M
    return (c / alpha) / DEG


C3 = float(_adj(M) * math.factorial(M - 1))
C2 = float(_adj(2))

NC = 2
NS = 16
NW = NC * NS
ROWS_PER_T = 1 + D2 + D3 * (M - 1)
CHUNK_T = 4
CHUNK_I = CHUNK_T * ROWS_PER_T
T_PER_W = B // NW
NITER = T_PER_W // CHUNK_T
NBUF = 2
LANES = 16


@functools.partial(
    pl.kernel,
    mesh=plsc.VectorSubcoreMesh(core_axis_name="c", subcore_axis_name="s"),
    out_type=jax.ShapeDtypeStruct((B, D_FEAT), jnp.float32),
    scratch_types=[
        pltpu.VMEM((NITER, CHUNK_I), jnp.int32),
        pltpu.VMEM((CHUNK_I, D_FEAT), jnp.float32),
        pltpu.VMEM((CHUNK_I, D_FEAT), jnp.float32),
        pltpu.VMEM((CHUNK_T, D_FEAT), jnp.float32),
        pltpu.SemaphoreType.DMA,
        pltpu.SemaphoreType.DMA,
    ],
)
def _sc_run(feat_hbm, idx_hbm, out_hbm, idx_v, rows0, rows1, out_v, sem0, sem1):
    cid = lax.axis_index("c")
    sid = lax.axis_index("s")
    wid = sid * NC + cid
    base_t = wid * T_PER_W

    pltpu.sync_copy(idx_hbm.at[wid], idx_v)

    bufs = (rows0, rows1)
    sems = (sem0, sem1)

    def start(chunk, b):
        pltpu.async_copy(feat_hbm.at[idx_v.at[chunk]], bufs[b], sems[b])

    def wait(b):
        pltpu.make_async_copy(feat_hbm.at[idx_v.at[0]], bufs[b], sems[b]).wait()

    start(0, 0)
    start(1, 1)

    def outer(it, carry):
        for b in range(NBUF):
            chunk = it * NBUF + b
            wait(b)
            rows = bufs[b]

            for t in range(CHUNK_T):
                r0 = t * ROWS_PER_T

                def col_body(j, carry2, rows=rows, r0=r0, t=t):
                    sl = pl.ds(j * LANES, LANES)
                    ft = rows[r0, sl]
                    ft2 = ft + ft
                    s2 = None
                    for e in range(D2):
                        fo = rows[r0 + 1 + e, sl]
                        term = (ft2 + fo) * fo
                        s2 = term if s2 is None else s2 + term
                    s3 = None
                    for e in range(D3):
                        pa = rows[r0 + 1 + D2 + 2 * e, sl]
                        pb = rows[r0 + 2 + D2 + 2 * e, sl]
                        prod = pa * pb
                        s3 = prod if s3 is None else s3 + prod
                    out_v[t, sl] = s2 * jnp.float32(C2) + s3 * jnp.float32(C3)
                    return carry2

                lax.fori_loop(0, D_FEAT // LANES, col_body, 0)

            pltpu.sync_copy(out_v, out_hbm.at[pl.ds(base_t + chunk * CHUNK_T, CHUNK_T)])

            nxt = chunk + NBUF

            @pl.when(nxt < NITER)
            def _():
                start(nxt, b)

        return carry

    lax.fori_loop(0, NITER // NBUF, outer, 0)


def kernel(target_nodes, features, edge3_others, edge2_others):
    idx = jnp.concatenate(
        [
            target_nodes[:, None],
            edge2_others,
            edge3_others.reshape(B, D3 * (M - 1)),
        ],
        axis=1,
    ).astype(jnp.int32)
    idx = idx.reshape(NW, NITER, CHUNK_I)
    return _sc_run(features, idx)
